# baseline (speedup 1.0000x reference)
import jax
import jax.numpy as jnp
from jax import lax
from jax.experimental import pallas as pl
from jax.experimental.pallas import tpu as pltpu
from jax.experimental.pallas import tpu_sc as plsc

B = 16384
D = 64
NC = 2
NS = 16
NW = NC * NS
BPW = B // NW
K = 128
CH = BPW // K


def _half_body(idx_hbm, tbl_hbm, dense_hbm, out_hbm,
               idx_v, rows_v, dense_v, semg, semd):
    wid = lax.axis_index("s") * NC + lax.axis_index("c")
    pltpu.sync_copy(idx_hbm.at[wid], idx_v)
    for j in range(CH):
        cb = wid * BPW + j * K
        g = pltpu.async_copy(tbl_hbm.at[idx_v.at[j]], rows_v, semg)
        c = pltpu.async_copy(dense_hbm.at[pl.ds(cb, K)], dense_v, semd)
        g.wait()
        pltpu.sync_copy(rows_v, out_hbm.at[pl.ds(cb, K), 0])
        c.wait()
        pltpu.sync_copy(dense_v, out_hbm.at[pl.ds(cb, K), 1])


def _make_half():
    return pl.kernel(
        _half_body,
        mesh=plsc.VectorSubcoreMesh(core_axis_name="c", subcore_axis_name="s"),
        compiler_params=pltpu.CompilerParams(use_tc_tiling_on_sc=False),
        out_type=jax.ShapeDtypeStruct((B, 2, D), jnp.float32),
        scratch_types=[
            pltpu.VMEM((CH, K), jnp.int32),
            pltpu.VMEM((K, D), jnp.float32),
            pltpu.VMEM((K, D), jnp.float32),
            pltpu.SemaphoreType.DMA,
            pltpu.SemaphoreType.DMA,
        ],
    )


_half_a = _make_half()
_half_b = _make_half()


@jax.jit
def kernel(adj_list_4, adj_list_5, concat_vecs_0, concat_vecs_1,
           concat_vecs_2, concat_vecs_3):
    idx4 = adj_list_4.astype(jnp.int32).reshape(NW, CH, K)
    idx5 = adj_list_5.astype(jnp.int32).reshape(NW, CH, K)
    pa = _half_a(idx5, concat_vecs_2, concat_vecs_0)
    pb = _half_b(idx4, concat_vecs_1, concat_vecs_3)
    return jnp.concatenate([pa.reshape(B, 2 * D), pb.reshape(B, 2 * D)],
                           axis=1)


# speedup vs baseline: 1.5380x; 1.5380x over previous
import jax
import jax.numpy as jnp
from jax import lax
from jax.experimental import pallas as pl
from jax.experimental.pallas import tpu as pltpu
from jax.experimental.pallas import tpu_sc as plsc

B = 16384
D = 64
VOCAB = 1000000
NC = 2
NS = 16
NW = NC * NS
BPW = B // NW
K = 128
CH = BPW // K
CW = 512
FULL_CHUNKS = 1953
TAIL_MULT = 32256
IPAD = B + 8
CAP = B // 16


def _p1_body(idx5_hbm, idx4_hbm, c2t, c1t, ri_hbm, ru_hbm,
             idxf_v, ci_v, cj_v, chunk_v, rows_v, jb_v, tmpc_v, tmpj_v):
    wid = lax.axis_index("s") * NC + lax.axis_index("c")
    start = jnp.where(wid == 0, 0, 62 + (wid - 1) * 61)
    nch = jnp.where(wid == 0, 62, 61)
    col_a = start * CW
    col_b = col_a + nch * CW + jnp.where(wid == 31, 128, 0)
    lane = lax.iota(jnp.int32, 16)
    dumpv = jnp.full((16,), B, jnp.int32)

    def init_jb():
        for k in range(K // 16):
            jb_v[pl.ds(16 * k, 16)] = dumpv

    def compress():
        def body(g, cnt):
            iv = idxf_v[pl.ds(g * 16, 16)]
            m = (iv >= col_a) & (iv < col_b)
            pos = lane * CAP + cnt
            plsc.store_scatter(ci_v, [pos], iv, mask=m)
            plsc.store_scatter(cj_v, [pos], g * 16 + lane, mask=m)
            return cnt + m.astype(jnp.int32)
        return pl.loop(0, B // 16,
                       init_carry=jnp.zeros((16,), jnp.int32))(body)

    def tree_max(v):
        for sh in (8, 4, 2, 1):
            tmpc_v[...] = v
            g = plsc.load_gather(tmpc_v, [jnp.minimum(lane + sh, 15)])
            v = jnp.maximum(v, g)
        return v

    def process(col0, cw, smax, cntv, fill, inter):
        def scan_body(s, fill):
            sv = jnp.full((16,), s, jnp.int32)
            e = plsc.load_gather(ci_v, [lane * CAP + sv])
            m = (sv < cntv) & (e >= col0) & (e < col0 + cw)
            cnt = plsc.all_reduce_population_count(m)[0]
            ej = plsc.load_gather(cj_v, [lane * CAP + sv])
            tmpc_v[...] = e - col0
            tmpj_v[...] = ej

            def member(_, carry):
                m, fill = carry
                t = plsc.all_reduce_ffs(m)
                lc = plsc.load_gather(tmpc_v, [t])
                jv = plsc.load_gather(tmpj_v, [t])
                fillv = jnp.full((16,), fill, jnp.int32)
                for k in range(4):
                    val = plsc.load_gather(chunk_v, [lane + 16 * k, lc])
                    plsc.store_scatter(rows_v, [fillv, lane + 16 * k], val)
                plsc.store_scatter(jb_v, [fillv], jv)
                m = m & (lane != t)
                fill = fill + 1

                @pl.when(fill == K)
                def _():
                    pltpu.sync_copy(rows_v, inter.at[jb_v])
                    init_jb()

                return (m, jnp.where(fill == K, 0, fill))

            _, fill = pl.loop(0, cnt, init_carry=(m, fill))(member)
            return fill

        return pl.loop(0, smax, init_carry=fill)(scan_body)

    for idx_hbm, tbl, inter in ((idx5_hbm, c2t, ri_hbm),
                                (idx4_hbm, c1t, ru_hbm)):
        pltpu.sync_copy(idx_hbm, idxf_v)
        cntv = compress()
        smax = tree_max(cntv)[0]
        init_jb()

        def chunk_body(k, fill):
            col0 = pl.multiple_of((start + k) * CW, 128)
            pltpu.sync_copy(tbl.at[:, pl.ds(col0, CW)], chunk_v)
            return process(col0, CW, smax, cntv, fill, inter)

        fill = pl.loop(0, nch, init_carry=jnp.int32(0))(chunk_body)

        def tail_body(_, fill):
            col0 = pl.multiple_of(wid * TAIL_MULT, 128)
            pltpu.sync_copy(tbl.at[:, pl.ds(col0, 128)],
                            chunk_v.at[:, pl.ds(0, 128)])
            return process(col0, 128, smax, cntv, fill, inter)

        fill = pl.loop(0, jnp.where(wid == 31, 1, 0),
                       init_carry=fill)(tail_body)

        pltpu.sync_copy(rows_v, inter.at[jb_v])


_p1 = pl.kernel(
    _p1_body,
    mesh=plsc.VectorSubcoreMesh(core_axis_name="c", subcore_axis_name="s"),
    compiler_params=pltpu.CompilerParams(needs_layout_passes=False),
    out_type=(jax.ShapeDtypeStruct((IPAD, 2 * D), jnp.float32),
              jax.ShapeDtypeStruct((IPAD, 2 * D), jnp.float32)),
    scratch_types=[
        pltpu.VMEM((B,), jnp.int32),
        pltpu.VMEM((B,), jnp.int32),
        pltpu.VMEM((B,), jnp.int32),
        pltpu.VMEM((D, CW), jnp.float32),
        pltpu.VMEM((K, 2 * D), jnp.float32),
        pltpu.VMEM((K,), jnp.int32),
        pltpu.VMEM((16,), jnp.int32),
        pltpu.VMEM((16,), jnp.int32),
    ],
)


def _p2_body(ri_hbm, ru_hbm, c0t, c3t, out_hbm, d0_v, d3_v, out_blk):
    wid = lax.axis_index("s") * NC + lax.axis_index("c")
    base = wid * BPW
    lane = lax.iota(jnp.int32, 16)

    def dense_extract(dsrc, qoff):
        @pl.loop(0, K)
        def body(j):
            colv = jnp.full((16,), j, jnp.int32)
            for k in range(4):
                v = plsc.load_gather(dsrc, [lane + 16 * k, colv])
                plsc.store_scatter(out_blk, [colv, (qoff + 16 * k) + lane], v)

    for jt in range(CH):
        cb = pl.multiple_of(base + jt * K, 128)
        pltpu.sync_copy(ri_hbm.at[pl.ds(cb, K)], out_blk.at[:, pl.ds(0, 2 * D)])
        pltpu.sync_copy(ru_hbm.at[pl.ds(cb, K)],
                        out_blk.at[:, pl.ds(2 * D, 2 * D)])
        pltpu.sync_copy(c0t.at[:, pl.ds(cb, K)], d0_v)
        pltpu.sync_copy(c3t.at[:, pl.ds(cb, K)], d3_v)
        dense_extract(d0_v, D)
        dense_extract(d3_v, 3 * D)
        pltpu.sync_copy(out_blk, out_hbm.at[pl.ds(cb, K)])


_p2 = pl.kernel(
    _p2_body,
    mesh=plsc.VectorSubcoreMesh(core_axis_name="c", subcore_axis_name="s"),
    compiler_params=pltpu.CompilerParams(needs_layout_passes=False),
    out_type=jax.ShapeDtypeStruct((B, 4 * D), jnp.float32),
    scratch_types=[
        pltpu.VMEM((D, K), jnp.float32),
        pltpu.VMEM((D, K), jnp.float32),
        pltpu.VMEM((K, 4 * D), jnp.float32),
    ],
)


@jax.jit
def kernel(adj_list_4, adj_list_5, concat_vecs_0, concat_vecs_1,
           concat_vecs_2, concat_vecs_3):
    a4 = adj_list_4.astype(jnp.int32)
    a5 = adj_list_5.astype(jnp.int32)
    ri, ru = _p1(a5, a4, concat_vecs_2.T, concat_vecs_1.T)
    return _p2(ri, ru, concat_vecs_0.T, concat_vecs_3.T)


# speedup vs baseline: 1.7753x; 1.1543x over previous
import jax
import jax.numpy as jnp
from jax import lax
from jax.experimental import pallas as pl
from jax.experimental.pallas import tpu as pltpu
from jax.experimental.pallas import tpu_sc as plsc

B = 16384
D = 64
VOCAB = 1000000
NC = 2
NS = 16
NW = NC * NS
BPW = B // NW
K = 128
CH = BPW // K
CW = 512
FULL_CHUNKS = 1953
TAIL_MULT = 32256
IPAD = B + 8
CAP = B // 16
KB = 96


def _p1_body(idx5_hbm, idx4_hbm, c2t, c1t, ri_hbm, ru_hbm,
             idxf_v, ci_v, cj_v, chunk_v, chunk2_v, rows_v, jb_v,
             tmpc_v, tmpj_v, sem0, sem1):
    wid = lax.axis_index("s") * NC + lax.axis_index("c")
    start = jnp.where(wid == 0, 0, 62 + (wid - 1) * 61)
    nch = jnp.where(wid == 0, 62, 61)
    col_a = start * CW
    col_b = col_a + nch * CW + jnp.where(wid == 31, 128, 0)
    lane = lax.iota(jnp.int32, 16)
    dumpv = jnp.full((16,), B, jnp.int32)

    def init_jb():
        for k in range(KB // 16):
            jb_v[pl.ds(16 * k, 16)] = dumpv

    def compress():
        def body(g, cnt):
            iv = idxf_v[pl.ds(g * 16, 16)]
            m = (iv >= col_a) & (iv < col_b)
            pos = lane * CAP + cnt
            plsc.store_scatter(ci_v, [pos], iv, mask=m)
            plsc.store_scatter(cj_v, [pos], g * 16 + lane, mask=m)
            return cnt + m.astype(jnp.int32)
        return pl.loop(0, B // 16,
                       init_carry=jnp.zeros((16,), jnp.int32))(body)

    def tree_max(v):
        for sh in (8, 4, 2, 1):
            tmpc_v[...] = v
            g = plsc.load_gather(tmpc_v, [jnp.minimum(lane + sh, 15)])
            v = jnp.maximum(v, g)
        return v

    def process(col0, cw, smax, cntv, fill, inter, chunk):
        def scan_body(s, fill):
            sv = jnp.full((16,), s, jnp.int32)
            e = plsc.load_gather(ci_v, [lane * CAP + sv])
            m = (sv < cntv) & (e >= col0) & (e < col0 + cw)
            cnt = plsc.all_reduce_population_count(m)[0]

            @pl.when(cnt > 0)
            def _():
                ej = plsc.load_gather(cj_v, [lane * CAP + sv])
                tmpc_v[...] = e - col0
                tmpj_v[...] = ej

            def member(_, carry):
                m, fill = carry
                t = plsc.all_reduce_ffs(m)
                lc = plsc.load_gather(tmpc_v, [t])
                jv = plsc.load_gather(tmpj_v, [t])
                fillv = jnp.full((16,), fill, jnp.int32)
                for k in range(4):
                    val = plsc.load_gather(chunk, [lane + 16 * k, lc])
                    plsc.store_scatter(rows_v, [fillv, lane + 16 * k], val)
                plsc.store_scatter(jb_v, [fillv], jv)
                m = m & (lane != t)
                fill = fill + 1

                @pl.when(fill == KB)
                def _():
                    pltpu.sync_copy(rows_v, inter.at[jb_v])
                    init_jb()

                return (m, jnp.where(fill == KB, 0, fill))

            _, fill = pl.loop(0, cnt, init_carry=(m, fill))(member)
            return fill

        return pl.loop(0, smax, init_carry=fill)(scan_body)

    for idx_hbm, tbl, inter in ((idx5_hbm, c2t, ri_hbm),
                                (idx4_hbm, c1t, ru_hbm)):
        pltpu.sync_copy(idx_hbm, idxf_v)
        cntv = compress()
        smax = tree_max(cntv)[0]
        init_jb()

        def pair_body(p, fill):
            k0 = p * 2
            c0 = pl.multiple_of((start + k0) * CW, 128)
            c1 = pl.multiple_of((start + k0 + 1) * CW, 128)
            h0 = pltpu.async_copy(tbl.at[:, pl.ds(c0, CW)], chunk_v, sem0)
            h1 = pltpu.async_copy(tbl.at[:, pl.ds(c1, CW)], chunk2_v, sem1)
            h0.wait()
            fill = process(c0, CW, smax, cntv, fill, inter, chunk_v)
            h1.wait()
            return process(c1, CW, smax, cntv, fill, inter, chunk2_v)

        fill = pl.loop(0, nch // 2, init_carry=jnp.int32(0))(pair_body)

        def odd_body(_, fill):
            col0 = pl.multiple_of((start + nch - 1) * CW, 128)
            pltpu.sync_copy(tbl.at[:, pl.ds(col0, CW)], chunk_v)
            return process(col0, CW, smax, cntv, fill, inter, chunk_v)

        fill = pl.loop(0, nch & 1, init_carry=fill)(odd_body)

        def tail_body(_, fill):
            col0 = pl.multiple_of(wid * TAIL_MULT, 128)
            pltpu.sync_copy(tbl.at[:, pl.ds(col0, 128)],
                            chunk_v.at[:, pl.ds(0, 128)])
            return process(col0, 128, smax, cntv, fill, inter, chunk_v)

        fill = pl.loop(0, jnp.where(wid == 31, 1, 0),
                       init_carry=fill)(tail_body)

        pltpu.sync_copy(rows_v, inter.at[jb_v])


_p1 = pl.kernel(
    _p1_body,
    mesh=plsc.VectorSubcoreMesh(core_axis_name="c", subcore_axis_name="s"),
    compiler_params=pltpu.CompilerParams(needs_layout_passes=False),
    out_type=(jax.ShapeDtypeStruct((IPAD, 2 * D), jnp.float32),
              jax.ShapeDtypeStruct((IPAD, 2 * D), jnp.float32)),
    scratch_types=[
        pltpu.VMEM((B,), jnp.int32),
        pltpu.VMEM((B,), jnp.int32),
        pltpu.VMEM((B,), jnp.int32),
        pltpu.VMEM((D, CW), jnp.float32),
        pltpu.VMEM((D, CW), jnp.float32),
        pltpu.VMEM((KB, 2 * D), jnp.float32),
        pltpu.VMEM((KB,), jnp.int32),
        pltpu.VMEM((16,), jnp.int32),
        pltpu.VMEM((16,), jnp.int32),
        pltpu.SemaphoreType.DMA,
        pltpu.SemaphoreType.DMA,
    ],
)


def _p2_body(ri_hbm, ru_hbm, c0t, c3t, out_hbm, d0_v, d3_v, out_blk,
             sem0, sem1, sem2, sem3):
    wid = lax.axis_index("s") * NC + lax.axis_index("c")
    base = wid * BPW
    lane = lax.iota(jnp.int32, 16)

    def dense_extract(dsrc, qoff):
        @pl.loop(0, K)
        def body(j):
            colv = jnp.full((16,), j, jnp.int32)
            for k in range(4):
                v = plsc.load_gather(dsrc, [lane + 16 * k, colv])
                plsc.store_scatter(out_blk, [colv, (qoff + 16 * k) + lane], v)

    for jt in range(CH):
        cb = pl.multiple_of(base + jt * K, 128)
        h0 = pltpu.async_copy(c0t.at[:, pl.ds(cb, K)], d0_v, sem0)
        h3 = pltpu.async_copy(c3t.at[:, pl.ds(cb, K)], d3_v, sem1)
        hi = pltpu.async_copy(ri_hbm.at[pl.ds(cb, K)],
                              out_blk.at[:, pl.ds(0, 2 * D)], sem2)
        hu = pltpu.async_copy(ru_hbm.at[pl.ds(cb, K)],
                              out_blk.at[:, pl.ds(2 * D, 2 * D)], sem3)
        h0.wait()
        hi.wait()
        dense_extract(d0_v, D)
        h3.wait()
        hu.wait()
        dense_extract(d3_v, 3 * D)
        pltpu.sync_copy(out_blk, out_hbm.at[pl.ds(cb, K)])


_p2 = pl.kernel(
    _p2_body,
    mesh=plsc.VectorSubcoreMesh(core_axis_name="c", subcore_axis_name="s"),
    compiler_params=pltpu.CompilerParams(needs_layout_passes=False),
    out_type=jax.ShapeDtypeStruct((B, 4 * D), jnp.float32),
    scratch_types=[
        pltpu.VMEM((D, K), jnp.float32),
        pltpu.VMEM((D, K), jnp.float32),
        pltpu.VMEM((K, 4 * D), jnp.float32),
        pltpu.SemaphoreType.DMA,
        pltpu.SemaphoreType.DMA,
        pltpu.SemaphoreType.DMA,
        pltpu.SemaphoreType.DMA,
    ],
)


@jax.jit
def kernel(adj_list_4, adj_list_5, concat_vecs_0, concat_vecs_1,
           concat_vecs_2, concat_vecs_3):
    a4 = adj_list_4.astype(jnp.int32)
    a5 = adj_list_5.astype(jnp.int32)
    ri, ru = _p1(a5, a4, concat_vecs_2.T, concat_vecs_1.T)
    return _p2(ri, ru, concat_vecs_0.T, concat_vecs_3.T)


# speedup vs baseline: 2.1050x; 1.1858x over previous
import jax
import jax.numpy as jnp
from jax import lax
from jax.experimental import pallas as pl
from jax.experimental.pallas import tpu as pltpu
from jax.experimental.pallas import tpu_sc as plsc

B = 16384
D = 64
VOCAB = 1000000
NC = 2
NS = 16
NW = NC * NS
BPW = B // NW
K = 128
CH = BPW // K
CW = 512
FULL_CHUNKS = 1953
TAIL_MULT = 32256
IPAD = B + 8
CAP = B // 16
KB = 96


def _p1_body(idx5_hbm, idx4_hbm, c2t, c1t, ri_hbm, ru_hbm,
             idxf_v, ci_v, cj_v, chunk_v, chunk2_v, rows_v, jb_v,
             tmpc_v, tmpj_v, sem0, sem1):
    wid = lax.axis_index("s") * NC + lax.axis_index("c")
    start = jnp.where(wid == 0, 0, 62 + (wid - 1) * 61)
    nch = jnp.where(wid == 0, 62, 61)
    col_a = start * CW
    col_b = col_a + nch * CW + jnp.where(wid == 31, 128, 0)
    lane = lax.iota(jnp.int32, 16)
    dumpv = jnp.full((16,), B, jnp.int32)

    def init_jb():
        for k in range(KB // 16):
            jb_v[pl.ds(16 * k, 16)] = dumpv

    def compress():
        def body(g, cnt):
            iv = idxf_v[pl.ds(g * 16, 16)]
            m = (iv >= col_a) & (iv < col_b)
            pos = lane * CAP + cnt
            plsc.store_scatter(ci_v, [pos], iv, mask=m)
            plsc.store_scatter(cj_v, [pos], g * 16 + lane, mask=m)
            return cnt + m.astype(jnp.int32)
        return pl.loop(0, B // 16,
                       init_carry=jnp.zeros((16,), jnp.int32))(body)

    def tree_max(v):
        for sh in (8, 4, 2, 1):
            tmpc_v[...] = v
            g = plsc.load_gather(tmpc_v, [jnp.minimum(lane + sh, 15)])
            v = jnp.maximum(v, g)
        return v

    def process(col0, cw, smax, cntv, fill, inter, chunk):
        def scan_body(s, fill):
            sv = jnp.full((16,), s, jnp.int32)
            e = plsc.load_gather(ci_v, [lane * CAP + sv])
            m = (sv < cntv) & (e >= col0) & (e < col0 + cw)
            cnt = plsc.all_reduce_population_count(m)[0]

            @pl.when(cnt > 0)
            def _():
                ej = plsc.load_gather(cj_v, [lane * CAP + sv])
                tmpc_v[...] = e - col0
                tmpj_v[...] = ej

            def member(_, carry):
                m, fill = carry
                t = plsc.all_reduce_ffs(m)
                lc = plsc.load_gather(tmpc_v, [t])
                jv = plsc.load_gather(tmpj_v, [t])
                fillv = jnp.full((16,), fill, jnp.int32)
                for k in range(4):
                    val = plsc.load_gather(chunk, [lane + 16 * k, lc])
                    plsc.store_scatter(rows_v, [fillv, lane + 16 * k], val)
                plsc.store_scatter(jb_v, [fillv], jv)
                m = m & (lane != t)
                fill = fill + 1

                @pl.when(fill == KB)
                def _():
                    pltpu.sync_copy(rows_v, inter.at[jb_v])
                    init_jb()

                return (m, jnp.where(fill == KB, 0, fill))

            _, fill = pl.loop(0, cnt, init_carry=(m, fill))(member)
            return fill

        return pl.loop(0, smax, init_carry=fill)(scan_body)

    for idx_hbm, tbl, inter in ((idx5_hbm, c2t, ri_hbm),
                                (idx4_hbm, c1t, ru_hbm)):
        pltpu.sync_copy(idx_hbm, idxf_v)
        cntv = compress()
        smax = tree_max(cntv)[0]
        init_jb()

        def cidx(k):
            return pl.multiple_of(
                (start + jnp.minimum(k, nch - 1)) * CW, 128)

        def wait_chunk(buf, sem):
            pltpu.make_async_copy(tbl.at[:, pl.ds(0, CW)], buf, sem).wait()

        pltpu.async_copy(tbl.at[:, pl.ds(cidx(0), CW)], chunk_v, sem0)
        pltpu.async_copy(tbl.at[:, pl.ds(cidx(1), CW)], chunk2_v, sem1)

        def pair_body(p, fill):
            k0 = p * 2
            c0 = pl.multiple_of((start + k0) * CW, 128)
            c1 = pl.multiple_of((start + k0 + 1) * CW, 128)
            wait_chunk(chunk_v, sem0)
            fill = process(c0, CW, smax, cntv, fill, inter, chunk_v)
            pltpu.async_copy(tbl.at[:, pl.ds(cidx(k0 + 2), CW)],
                             chunk_v, sem0)
            wait_chunk(chunk2_v, sem1)
            fill = process(c1, CW, smax, cntv, fill, inter, chunk2_v)
            pltpu.async_copy(tbl.at[:, pl.ds(cidx(k0 + 3), CW)],
                             chunk2_v, sem1)
            return fill

        fill = pl.loop(0, nch // 2, init_carry=jnp.int32(0))(pair_body)

        def odd_body(_, fill):
            wait_chunk(chunk_v, sem0)
            col0 = pl.multiple_of((start + nch - 1) * CW, 128)
            return process(col0, CW, smax, cntv, fill, inter, chunk_v)

        fill = pl.loop(0, nch & 1, init_carry=fill)(odd_body)

        def drain_a(_, c):
            wait_chunk(chunk_v, sem0)
            return c
        pl.loop(0, 1 - (nch & 1), init_carry=jnp.int32(0))(drain_a)
        wait_chunk(chunk2_v, sem1)

        def tail_body(_, fill):
            col0 = pl.multiple_of(wid * TAIL_MULT, 128)
            pltpu.sync_copy(tbl.at[:, pl.ds(col0, 128)],
                            chunk_v.at[:, pl.ds(0, 128)])
            return process(col0, 128, smax, cntv, fill, inter, chunk_v)

        fill = pl.loop(0, jnp.where(wid == 31, 1, 0),
                       init_carry=fill)(tail_body)

        pltpu.sync_copy(rows_v, inter.at[jb_v])


_p1 = pl.kernel(
    _p1_body,
    mesh=plsc.VectorSubcoreMesh(core_axis_name="c", subcore_axis_name="s"),
    compiler_params=pltpu.CompilerParams(needs_layout_passes=False),
    out_type=(jax.ShapeDtypeStruct((IPAD, 2 * D), jnp.float32),
              jax.ShapeDtypeStruct((IPAD, 2 * D), jnp.float32)),
    scratch_types=[
        pltpu.VMEM((B,), jnp.int32),
        pltpu.VMEM((B,), jnp.int32),
        pltpu.VMEM((B,), jnp.int32),
        pltpu.VMEM((D, CW), jnp.float32),
        pltpu.VMEM((D, CW), jnp.float32),
        pltpu.VMEM((KB, 2 * D), jnp.float32),
        pltpu.VMEM((KB,), jnp.int32),
        pltpu.VMEM((16,), jnp.int32),
        pltpu.VMEM((16,), jnp.int32),
        pltpu.SemaphoreType.DMA,
        pltpu.SemaphoreType.DMA,
    ],
)


def _p2_body(ri_hbm, ru_hbm, c0t, c3t, out_hbm, d0_v, d3_v, out_blk,
             sem0, sem1, sem2, sem3):
    wid = lax.axis_index("s") * NC + lax.axis_index("c")
    base = wid * BPW
    lane = lax.iota(jnp.int32, 16)

    def dense_extract(dsrc, qoff):
        @pl.loop(0, K)
        def body(j):
            colv = jnp.full((16,), j, jnp.int32)
            for k in range(4):
                v = plsc.load_gather(dsrc, [lane + 16 * k, colv])
                plsc.store_scatter(out_blk, [colv, (qoff + 16 * k) + lane], v)

    for jt in range(CH):
        cb = pl.multiple_of(base + jt * K, 128)
        h0 = pltpu.async_copy(c0t.at[:, pl.ds(cb, K)], d0_v, sem0)
        h3 = pltpu.async_copy(c3t.at[:, pl.ds(cb, K)], d3_v, sem1)
        hi = pltpu.async_copy(ri_hbm.at[pl.ds(cb, K)],
                              out_blk.at[:, pl.ds(0, 2 * D)], sem2)
        hu = pltpu.async_copy(ru_hbm.at[pl.ds(cb, K)],
                              out_blk.at[:, pl.ds(2 * D, 2 * D)], sem3)
        h0.wait()
        hi.wait()
        dense_extract(d0_v, D)
        h3.wait()
        hu.wait()
        dense_extract(d3_v, 3 * D)
        pltpu.sync_copy(out_blk, out_hbm.at[pl.ds(cb, K)])


_p2 = pl.kernel(
    _p2_body,
    mesh=plsc.VectorSubcoreMesh(core_axis_name="c", subcore_axis_name="s"),
    compiler_params=pltpu.CompilerParams(needs_layout_passes=False),
    out_type=jax.ShapeDtypeStruct((B, 4 * D), jnp.float32),
    scratch_types=[
        pltpu.VMEM((D, K), jnp.float32),
        pltpu.VMEM((D, K), jnp.float32),
        pltpu.VMEM((K, 4 * D), jnp.float32),
        pltpu.SemaphoreType.DMA,
        pltpu.SemaphoreType.DMA,
        pltpu.SemaphoreType.DMA,
        pltpu.SemaphoreType.DMA,
    ],
)


@jax.jit
def kernel(adj_list_4, adj_list_5, concat_vecs_0, concat_vecs_1,
           concat_vecs_2, concat_vecs_3):
    a4 = adj_list_4.astype(jnp.int32)
    a5 = adj_list_5.astype(jnp.int32)
    ri, ru = _p1(a5, a4, concat_vecs_2.T, concat_vecs_1.T)
    return _p2(ri, ru, concat_vecs_0.T, concat_vecs_3.T)
